# baseline (speedup 1.0000x reference)
import functools

import jax
import jax.numpy as jnp
from jax import lax
from jax.experimental import pallas as pl
from jax.experimental.pallas import tpu as pltpu
from jax.experimental.pallas import tpu_sc as plsc

N = 10000
E = 320000
D = 128

NC = 2
NS = 16
NW = NC * NS
EW = E // NW
C = 80
CH = EW // C
CHP = 128
EDR = CHP + 2
NP = 10240
RPS = NP // NS


def _sc_body(nfeat_hbm, edata_hbm, ew_hbm, zeros_hbm, out_hbm,
             ebuf0, ebuf1, wfb0, wfb1, dstb0, dstb1, wb0, wb1,
             rows0, rows1, hn_sh,
             esem0, esem1, wsem0, wsem1, gsem0, gsem1, ssem0, ssem1):
    c = lax.axis_index("c")
    s = lax.axis_index("s")
    wid = c * NS + s

    ebuf = (ebuf0, ebuf1)
    wfb = (wfb0, wfb1)
    dstb = (dstb0, dstb1)
    wb = (wb0, wb1)
    rows = (rows0, rows1)
    esem = (esem0, esem1)
    wsem = (wsem0, wsem1)
    gsem = (gsem0, gsem1)
    ssem = (ssem0, ssem1)

    pltpu.sync_copy(zeros_hbm.at[pl.ds(s * RPS, RPS)],
                    hn_sh.at[pl.ds(s * RPS, RPS)])

    plsc.subcore_barrier()

    def compute(b):
        rv = rows[b]
        wbuf = wb[b]

        def group(g, carry):
            w16 = wbuf[0, pl.ds(g * 16, 16)]
            for e16 in range(16):
                wv = jnp.full((16,), w16[e16], jnp.float32)
                e = g * 16 + e16
                for j in range(D // 16):
                    sl = pl.ds(j * 16, 16)
                    rv[e, sl] = rv[e, sl] * wv
            return carry

        lax.fori_loop(0, C // 16, group, 0)

    def phase(j, b, first=False):
        o = 1 - b
        if not first:
            pltpu.make_async_copy(rows[o], hn_sh.at[dstb[o].at[0]],
                                  ssem[o]).wait()
        pltpu.make_async_copy(edata_hbm.at[wid, j + 1], ebuf[o],
                              esem[o]).wait()
        pltpu.make_async_copy(ew_hbm.at[wid, j + 1], wfb[o], wsem[o]).wait()
        pltpu.async_copy(nfeat_hbm.at[ebuf[o].at[0]], rows[o], gsem[o])
        for g in range(C // 16):
            sl = pl.ds(g * 16, 16)
            dstb[b][0, sl] = ebuf[b][1, sl]
            wb[b][0, sl] = wfb[b][0, sl]
        pltpu.make_async_copy(nfeat_hbm.at[ebuf[b].at[0]], rows[b],
                              gsem[b]).wait()
        pltpu.async_copy(edata_hbm.at[wid, j + 2], ebuf[b], esem[b])
        pltpu.async_copy(ew_hbm.at[wid, j + 2], wfb[b], wsem[b])
        compute(b)
        pltpu.async_copy(rows[b], hn_sh.at[dstb[b].at[0]], ssem[b], add=True)

    pltpu.sync_copy(edata_hbm.at[wid, 0], ebuf0)
    pltpu.sync_copy(ew_hbm.at[wid, 0], wfb0)
    pltpu.async_copy(edata_hbm.at[wid, 1], ebuf1, esem1)
    pltpu.async_copy(ew_hbm.at[wid, 1], wfb1, wsem1)
    pltpu.async_copy(nfeat_hbm.at[ebuf0.at[0]], rows0, gsem0)

    phase(0, 0, first=True)
    phase(1, 1)

    def pair(k, carry):
        j = 2 * k + 2
        phase(j, 0)
        phase(j + 1, 1)
        return carry

    lax.fori_loop(0, (CHP - 2) // 2, pair, 0)

    pltpu.make_async_copy(nfeat_hbm.at[ebuf0.at[0]], rows0, gsem0).wait()
    pltpu.make_async_copy(edata_hbm.at[wid, EDR - 1], ebuf1, esem1).wait()
    pltpu.make_async_copy(ew_hbm.at[wid, EDR - 1], wfb1, wsem1).wait()
    pltpu.make_async_copy(rows1, hn_sh.at[dstb1.at[0]], ssem1).wait()

    plsc.subcore_barrier()

    pltpu.sync_copy(hn_sh.at[pl.ds(s * RPS, RPS)],
                    out_hbm.at[c, pl.ds(s * RPS, RPS)])


_sc_call = functools.partial(
    pl.kernel,
    out_type=jax.ShapeDtypeStruct((NC, NP, D), jnp.float32),
    mesh=plsc.VectorSubcoreMesh(core_axis_name="c", subcore_axis_name="s"),
    scratch_types=[
        pltpu.VMEM((2, C), jnp.int32),
        pltpu.VMEM((2, C), jnp.int32),
        pltpu.VMEM((1, C), jnp.float32),
        pltpu.VMEM((1, C), jnp.float32),
        pltpu.VMEM((1, C), jnp.int32),
        pltpu.VMEM((1, C), jnp.int32),
        pltpu.VMEM((1, C), jnp.float32),
        pltpu.VMEM((1, C), jnp.float32),
        pltpu.VMEM((C, D), jnp.float32),
        pltpu.VMEM((C, D), jnp.float32),
        pltpu.VMEM_SHARED((NP, D), jnp.float32),
        pltpu.SemaphoreType.DMA,
        pltpu.SemaphoreType.DMA,
        pltpu.SemaphoreType.DMA,
        pltpu.SemaphoreType.DMA,
        pltpu.SemaphoreType.DMA,
        pltpu.SemaphoreType.DMA,
        pltpu.SemaphoreType.DMA,
        pltpu.SemaphoreType.DMA,
    ],
)(_sc_body)


def _tc_body(h_ref, p_ref, w1_ref, w2_ref, hn_ref, out_ref):
    h = h_ref[...]
    hn = p_ref[0] + p_ref[1]
    hn_ref[...] = hn
    a = lax.dot_general(h + hn, w1_ref[...], (((1,), (1,)), ((), ())),
                        precision=lax.Precision.HIGHEST,
                        preferred_element_type=jnp.float32)
    b = lax.dot_general(h * hn, w2_ref[...], (((1,), (1,)), ((), ())),
                        precision=lax.Precision.HIGHEST,
                        preferred_element_type=jnp.float32)
    out_ref[...] = (jnp.where(a > 0, a, 0.01 * a)
                    + jnp.where(b > 0, b, 0.01 * b))


_TB = 1024

_tc_call = pl.pallas_call(
    _tc_body,
    grid=(pl.cdiv(N, _TB),),
    in_specs=[
        pl.BlockSpec((_TB, D), lambda i: (i, 0)),
        pl.BlockSpec((NC, _TB, D), lambda i: (0, i, 0)),
        pl.BlockSpec((D, D), lambda i: (0, 0)),
        pl.BlockSpec((D, D), lambda i: (0, 0)),
    ],
    out_specs=[
        pl.BlockSpec((_TB, D), lambda i: (i, 0)),
        pl.BlockSpec((_TB, D), lambda i: (i, 0)),
    ],
    out_shape=[
        jax.ShapeDtypeStruct((N, D), jnp.float32),
        jax.ShapeDtypeStruct((N, D), jnp.float32),
    ],
)


def kernel(nfeat, edge_index, edge_weight, W1, W2):
    pad = ((0, 0), (0, EDR - CH), (0, 0))
    src = jnp.pad(edge_index[0].astype(jnp.int32).reshape(NW, CH, C), pad)
    dst = jnp.pad(edge_index[1].astype(jnp.int32).reshape(NW, CH, C), pad)
    edata = jnp.stack([src, dst], axis=2)
    ew = jnp.pad(edge_weight.astype(jnp.float32).reshape(NW, CH, C),
                 pad)[:, :, None, :]
    zeros = jnp.zeros((NP, D), jnp.float32)
    partials = _sc_call(nfeat, edata, ew, zeros)
    hn, out = _tc_call(nfeat, partials, W1, W2)
    return (hn, out)


# speedup vs baseline: 1.0004x; 1.0004x over previous
import functools

import jax
import jax.numpy as jnp
from jax import lax
from jax.experimental import pallas as pl
from jax.experimental.pallas import tpu as pltpu
from jax.experimental.pallas import tpu_sc as plsc

N = 10000
E = 320000
D = 128

NC = 2
NS = 16
NW = NC * NS
EW = E // NW
C = 80
CH = EW // C
CHP = 128
EDR = CHP + 2
NP = 10240
RPS = NP // NS


def _sc_body(nfeat_hbm, edata_hbm, ew_hbm, zeros_hbm, out_hbm,
             ebuf0, ebuf1, wfb0, wfb1, dstb0, dstb1, wb0, wb1,
             rows0, rows1, hn_sh,
             esem0, esem1, wsem0, wsem1, gsem0, gsem1, ssem0, ssem1):
    c = lax.axis_index("c")
    s = lax.axis_index("s")
    wid = c * NS + s

    ebuf = (ebuf0, ebuf1)
    wfb = (wfb0, wfb1)
    dstb = (dstb0, dstb1)
    wb = (wb0, wb1)
    rows = (rows0, rows1)
    esem = (esem0, esem1)
    wsem = (wsem0, wsem1)
    gsem = (gsem0, gsem1)
    ssem = (ssem0, ssem1)

    pltpu.sync_copy(zeros_hbm.at[pl.ds(s * RPS, RPS)],
                    hn_sh.at[pl.ds(s * RPS, RPS)])

    plsc.subcore_barrier()

    def compute(b):
        rv = rows[b]
        wbuf = wb[b]

        def group(g, carry):
            w16 = wbuf[0, pl.ds(g * 16, 16)]
            for e16 in range(16):
                wv = jnp.full((16,), w16[e16], jnp.float32)
                e = g * 16 + e16
                for j in range(D // 16):
                    sl = pl.ds(j * 16, 16)
                    rv[e, sl] = rv[e, sl] * wv
            return carry

        lax.fori_loop(0, C // 16, group, 0)

    def phase(j, b, first=False):
        o = 1 - b
        pltpu.make_async_copy(edata_hbm.at[wid, j + 1], ebuf[o],
                              esem[o]).wait()
        pltpu.make_async_copy(ew_hbm.at[wid, j + 1], wfb[o], wsem[o]).wait()
        pltpu.async_copy(nfeat_hbm.at[ebuf[o].at[0]], rows[o], gsem[o])
        for g in range(C // 16):
            sl = pl.ds(g * 16, 16)
            dstb[b][0, sl] = ebuf[b][1, sl]
            wb[b][0, sl] = wfb[b][0, sl]
        pltpu.make_async_copy(nfeat_hbm.at[ebuf[b].at[0]], rows[b],
                              gsem[b]).wait()
        pltpu.async_copy(edata_hbm.at[wid, j + 2], ebuf[b], esem[b])
        pltpu.async_copy(ew_hbm.at[wid, j + 2], wfb[b], wsem[b])
        compute(b)
        pltpu.sync_copy(rows[b], hn_sh.at[dstb[b].at[0]], add=True)

    pltpu.sync_copy(edata_hbm.at[wid, 0], ebuf0)
    pltpu.sync_copy(ew_hbm.at[wid, 0], wfb0)
    pltpu.async_copy(edata_hbm.at[wid, 1], ebuf1, esem1)
    pltpu.async_copy(ew_hbm.at[wid, 1], wfb1, wsem1)
    pltpu.async_copy(nfeat_hbm.at[ebuf0.at[0]], rows0, gsem0)

    phase(0, 0, first=True)
    phase(1, 1)

    def pair(k, carry):
        j = 2 * k + 2
        phase(j, 0)
        phase(j + 1, 1)
        return carry

    lax.fori_loop(0, (CHP - 2) // 2, pair, 0)

    pltpu.make_async_copy(nfeat_hbm.at[ebuf0.at[0]], rows0, gsem0).wait()
    pltpu.make_async_copy(edata_hbm.at[wid, EDR - 1], ebuf1, esem1).wait()
    pltpu.make_async_copy(ew_hbm.at[wid, EDR - 1], wfb1, wsem1).wait()

    plsc.subcore_barrier()

    pltpu.sync_copy(hn_sh.at[pl.ds(s * RPS, RPS)],
                    out_hbm.at[c, pl.ds(s * RPS, RPS)])


_sc_call = functools.partial(
    pl.kernel,
    out_type=jax.ShapeDtypeStruct((NC, NP, D), jnp.float32),
    mesh=plsc.VectorSubcoreMesh(core_axis_name="c", subcore_axis_name="s"),
    scratch_types=[
        pltpu.VMEM((2, C), jnp.int32),
        pltpu.VMEM((2, C), jnp.int32),
        pltpu.VMEM((1, C), jnp.float32),
        pltpu.VMEM((1, C), jnp.float32),
        pltpu.VMEM((1, C), jnp.int32),
        pltpu.VMEM((1, C), jnp.int32),
        pltpu.VMEM((1, C), jnp.float32),
        pltpu.VMEM((1, C), jnp.float32),
        pltpu.VMEM((C, D), jnp.float32),
        pltpu.VMEM((C, D), jnp.float32),
        pltpu.VMEM_SHARED((NP, D), jnp.float32),
        pltpu.SemaphoreType.DMA,
        pltpu.SemaphoreType.DMA,
        pltpu.SemaphoreType.DMA,
        pltpu.SemaphoreType.DMA,
        pltpu.SemaphoreType.DMA,
        pltpu.SemaphoreType.DMA,
        pltpu.SemaphoreType.DMA,
        pltpu.SemaphoreType.DMA,
    ],
)(_sc_body)


def _tc_body(h_ref, p_ref, w1_ref, w2_ref, hn_ref, out_ref):
    h = h_ref[...]
    hn = p_ref[0] + p_ref[1]
    hn_ref[...] = hn
    a = lax.dot_general(h + hn, w1_ref[...], (((1,), (1,)), ((), ())),
                        precision=lax.Precision.HIGHEST,
                        preferred_element_type=jnp.float32)
    b = lax.dot_general(h * hn, w2_ref[...], (((1,), (1,)), ((), ())),
                        precision=lax.Precision.HIGHEST,
                        preferred_element_type=jnp.float32)
    out_ref[...] = (jnp.where(a > 0, a, 0.01 * a)
                    + jnp.where(b > 0, b, 0.01 * b))


_TB = 1024

_tc_call = pl.pallas_call(
    _tc_body,
    grid=(pl.cdiv(N, _TB),),
    in_specs=[
        pl.BlockSpec((_TB, D), lambda i: (i, 0)),
        pl.BlockSpec((NC, _TB, D), lambda i: (0, i, 0)),
        pl.BlockSpec((D, D), lambda i: (0, 0)),
        pl.BlockSpec((D, D), lambda i: (0, 0)),
    ],
    out_specs=[
        pl.BlockSpec((_TB, D), lambda i: (i, 0)),
        pl.BlockSpec((_TB, D), lambda i: (i, 0)),
    ],
    out_shape=[
        jax.ShapeDtypeStruct((N, D), jnp.float32),
        jax.ShapeDtypeStruct((N, D), jnp.float32),
    ],
)


def kernel(nfeat, edge_index, edge_weight, W1, W2):
    pad = ((0, 0), (0, EDR - CH), (0, 0))
    src = jnp.pad(edge_index[0].astype(jnp.int32).reshape(NW, CH, C), pad)
    dst = jnp.pad(edge_index[1].astype(jnp.int32).reshape(NW, CH, C), pad)
    edata = jnp.stack([src, dst], axis=2)
    ew = jnp.pad(edge_weight.astype(jnp.float32).reshape(NW, CH, C),
                 pad)[:, :, None, :]
    zeros = jnp.zeros((NP, D), jnp.float32)
    partials = _sc_call(nfeat, edata, ew, zeros)
    hn, out = _tc_call(nfeat, partials, W1, W2)
    return (hn, out)
